# baseline (speedup 1.0000x reference)
import functools

import jax
import jax.numpy as jnp
from jax import lax
from jax.experimental import pallas as pl
from jax.experimental.pallas import tpu as pltpu
from jax.experimental.pallas import tpu_sc as plsc

_NCH = 1


@functools.lru_cache(maxsize=None)
def _make_kernel(V, D, B, C):
    info = plsc.get_sparse_core_info()
    NC, NS, L = info.num_cores, info.num_subcores, info.num_lanes
    NW = NC * NS
    assert B % (NW * _NCH) == 0 and D % L == 0 and C % L == 0
    b_per_w = B // NW
    ch = b_per_w // _NCH
    mesh = plsc.VectorSubcoreMesh(core_axis_name="c", subcore_axis_name="s")

    @functools.partial(
        pl.kernel,
        mesh=mesh,
        out_type=jax.ShapeDtypeStruct((B, D), jnp.float32),
        scratch_types=[
            pltpu.VMEM((b_per_w,), jnp.int32),
            pltpu.VMEM((b_per_w, D), jnp.float32),
            pltpu.VMEM((1, C), jnp.float32),
            pltpu.SemaphoreType.DMA,
            pltpu.SemaphoreType.DMA,
        ],
    )
    def k(idx_hbm, table_hbm, shared_hbm, out_hbm, idx_v, rows_v, sh_v, sg, sw):
        wid = lax.axis_index("s") * NC + lax.axis_index("c")
        base = wid * b_per_w
        pltpu.sync_copy(idx_hbm.at[pl.ds(base, b_per_w)], idx_v)
        sh_cp = pltpu.async_copy(shared_hbm, sh_v, sw)
        gathers = [
            pltpu.async_copy(
                table_hbm.at[idx_v.at[pl.ds(g * ch, ch)]],
                rows_v.at[pl.ds(g * ch, ch)],
                sg,
            )
            for g in range(_NCH)
        ]
        sh_cp.wait()
        shared_regs = [sh_v[0, pl.ds(j * L, L)] for j in range(C // L)]
        writes = []
        for g in range(_NCH):
            gathers[g].wait()

            def fill(i, carry, g=g):
                for j in range(C // L):
                    rows_v[g * ch + i, pl.ds(j * L, L)] = shared_regs[j]
                return carry

            lax.fori_loop(0, ch, fill, 0)
            writes.append(
                pltpu.async_copy(
                    rows_v.at[pl.ds(g * ch, ch)],
                    out_hbm.at[pl.ds(base + g * ch, ch)],
                    sw,
                )
            )
        for w in writes:
            w.wait()

    return k


def kernel(X, table, shared_embed):
    idx = X.astype(jnp.int32)
    k = _make_kernel(
        table.shape[0], table.shape[1], X.shape[0], shared_embed.shape[1]
    )
    return k(idx, table, shared_embed)


# speedup vs baseline: 1.0832x; 1.0832x over previous
import functools

import jax
import jax.numpy as jnp
from jax import lax
from jax.experimental import pallas as pl
from jax.experimental.pallas import tpu as pltpu
from jax.experimental.pallas import tpu_sc as plsc

_NCH = 2


@functools.lru_cache(maxsize=None)
def _make_kernel(V, D, B, C):
    info = plsc.get_sparse_core_info()
    NC, NS, L = info.num_cores, info.num_subcores, info.num_lanes
    NW = NC * NS
    assert B % (NW * _NCH) == 0 and D % L == 0 and C % L == 0
    b_per_w = B // NW
    ch = b_per_w // _NCH
    mesh = plsc.VectorSubcoreMesh(core_axis_name="c", subcore_axis_name="s")

    @functools.partial(
        pl.kernel,
        mesh=mesh,
        out_type=jax.ShapeDtypeStruct((B, D), jnp.float32),
        scratch_types=[
            pltpu.VMEM((b_per_w,), jnp.int32),
            pltpu.VMEM((b_per_w, D), jnp.float32),
            pltpu.VMEM((1, C), jnp.float32),
            pltpu.SemaphoreType.DMA,
            pltpu.SemaphoreType.DMA,
        ],
    )
    def k(idx_hbm, table_hbm, shared_hbm, out_hbm, idx_v, rows_v, sh_v, sg, sw):
        wid = lax.axis_index("s") * NC + lax.axis_index("c")
        base = wid * b_per_w
        pltpu.sync_copy(idx_hbm.at[pl.ds(base, b_per_w)], idx_v)
        sh_cp = pltpu.async_copy(shared_hbm, sh_v, sw)
        gathers = [
            pltpu.async_copy(
                table_hbm.at[idx_v.at[pl.ds(g * ch, ch)]],
                rows_v.at[pl.ds(g * ch, ch)],
                sg,
            )
            for g in range(_NCH)
        ]
        sh_cp.wait()
        shared_regs = [sh_v[0, pl.ds(j * L, L)] for j in range(C // L)]
        writes = []
        for g in range(_NCH):
            gathers[g].wait()

            def fill(i, carry, g=g):
                for j in range(C // L):
                    rows_v[g * ch + i, pl.ds(j * L, L)] = shared_regs[j]
                return carry

            lax.fori_loop(0, ch, fill, 0)
            writes.append(
                pltpu.async_copy(
                    rows_v.at[pl.ds(g * ch, ch)],
                    out_hbm.at[pl.ds(base + g * ch, ch)],
                    sw,
                )
            )
        for w in writes:
            w.wait()

    return k


def kernel(X, table, shared_embed):
    idx = X.astype(jnp.int32)
    k = _make_kernel(
        table.shape[0], table.shape[1], X.shape[0], shared_embed.shape[1]
    )
    return k(idx, table, shared_embed)
